# Initial kernel scaffold; baseline (speedup 1.0000x reference)
import jax
import jax.numpy as jnp
from jax.experimental import pallas as pl


def kernel(cosine, label):
    raise NotImplementedError("write your pallas kernel here")


# speedup vs baseline: 1.1177x; 1.1177x over previous
import functools

import jax
import jax.numpy as jnp
from jax.experimental import pallas as pl

_A = 0.12
_B = 2.6
_C = 1.6
_S = 64.0

_BC = 2048


def _acos(x):
    ax = jnp.abs(x)
    p = jnp.float32(-0.0012624911)
    p = p * ax + jnp.float32(0.0066700901)
    p = p * ax + jnp.float32(-0.0170881256)
    p = p * ax + jnp.float32(0.0308918810)
    p = p * ax + jnp.float32(-0.0501743046)
    p = p * ax + jnp.float32(0.0889789874)
    p = p * ax + jnp.float32(-0.2145988016)
    p = p * ax + jnp.float32(1.5707963050)
    r = jnp.sqrt(jnp.maximum(1.0 - ax, 0.0)) * p
    return jnp.where(x >= 0, r, jnp.float32(3.14159265358979) - r)


def _tile_body(cos_ref, lab_ref, out_ref):
    j = pl.program_id(0)
    tile = cos_ref[...]
    lab = lab_ref[...]
    col0 = j * _BC
    cols = col0 + jax.lax.broadcasted_iota(jnp.int32, tile.shape, 1)
    hit = cols == lab
    orig = jnp.sum(jnp.where(hit, tile, 0.0), axis=1, keepdims=True)
    t = _acos(orig) + _B
    tgt = -_A * (t * t) + _C
    out_ref[...] = jnp.where(hit, tgt, tile) * _S


def kernel(cosine, label):
    batch, vocab = cosine.shape
    lab2 = label.reshape(batch, 1)
    grid = (pl.cdiv(vocab, _BC),)
    return pl.pallas_call(
        _tile_body,
        grid=grid,
        in_specs=[
            pl.BlockSpec((batch, _BC), lambda j: (0, j)),
            pl.BlockSpec((batch, 1), lambda j: (0, 0)),
        ],
        out_specs=pl.BlockSpec((batch, _BC), lambda j: (0, j)),
        out_shape=jax.ShapeDtypeStruct((batch, vocab), jnp.float32),
    )(cosine, lab2)
